# baseline (speedup 1.0000x reference)
import functools

import jax
import jax.numpy as jnp
from jax import lax
from jax.experimental import pallas as pl
from jax.experimental.pallas import tpu as pltpu
from jax.experimental.pallas import tpu_sc as plsc

NC = 2
NS = 16
NW = NC * NS
BZ = 256
L = 16


@functools.partial(jax.jit, static_argnames=("B", "F", "D"))
def _emb_lookup(idxT, W, B, F, D):
    n_blk = B // BZ
    n_units = F * n_blk
    u_per_w = n_units // NW
    assert n_units % NW == 0 and u_per_w % 2 == 0

    mesh = plsc.VectorSubcoreMesh(
        core_axis_name="c", subcore_axis_name="s",
        num_cores=NC, num_subcores=NS,
    )

    @functools.partial(
        pl.kernel,
        out_type=jax.ShapeDtypeStruct((F, D, B), jnp.float32),
        mesh=mesh,
        scratch_types=[
            [pltpu.VMEM((BZ,), jnp.int32)] * 2,
            [pltpu.VMEM((BZ, D), jnp.float32)] * 2,
            [pltpu.VMEM((D, BZ), jnp.float32)] * 2,
            [pltpu.SemaphoreType.DMA] * 2,
            [pltpu.SemaphoreType.DMA] * 2,
            [pltpu.SemaphoreType.DMA] * 2,
        ],
        compiler_params=pltpu.CompilerParams(
            use_tc_tiling_on_sc=False, needs_layout_passes=False
        ),
    )
    def k(idx_hbm, table_hbm, out_hbm, idxs, rows, bufTs, isems, gsems, osems):
        cid = lax.axis_index("c")
        sid = lax.axis_index("s")
        wid = sid * NC + cid
        u0 = wid * u_per_w

        def unit_fb(i):
            u = u0 + i
            f = lax.shift_right_logical(u, 6)
            b0 = pl.multiple_of(
                lax.shift_left(lax.bitwise_and(u, n_blk - 1), 8), BZ
            )
            return f, b0

        def issue_idx(i, p):
            f, b0 = unit_fb(i)
            pltpu.async_copy(idx_hbm.at[f, pl.ds(b0, BZ)], idxs[p], isems[p])

        def wait_idx(p):
            pltpu.make_async_copy(
                idx_hbm.at[0, pl.ds(0, BZ)], idxs[p], isems[p]
            ).wait()

        def issue_gather(p):
            for g in range(BZ // 128):
                pltpu.async_copy(
                    table_hbm.at[idxs[p].at[pl.ds(g * 128, 128)]],
                    rows[p].at[pl.ds(g * 128, 128)],
                    gsems[p],
                )

        def wait_gather(p):
            pltpu.make_async_copy(
                table_hbm.at[pl.ds(0, BZ)], rows[p], gsems[p]
            ).wait()

        def issue_out(i, p):
            f, b0 = unit_fb(i)
            pltpu.async_copy(
                bufTs[p], out_hbm.at[f, :, pl.ds(b0, BZ)], osems[p]
            )

        def wait_out(p):
            pltpu.make_async_copy(
                bufTs[p], out_hbm.at[0, :, pl.ds(0, BZ)], osems[p]
            ).wait()

        ii = jnp.arange(L, dtype=jnp.int32)
        row_ids = [ii + bl for bl in range(0, BZ, L)]

        def transpose(p):
            @pl.loop(0, D)
            def _(d):
                dd = jnp.full((L,), d, jnp.int32)
                for t, rid in enumerate(row_ids):
                    x = plsc.load_gather(rows[p], [rid, dd])
                    bufTs[p][d, pl.ds(t * L, L)] = x

        issue_idx(0, 0)
        wait_idx(0)
        issue_gather(0)
        issue_idx(1, 1)

        @pl.loop(0, u_per_w, step=2)
        def _(o):
            for b in range(2):
                i = o + b
                p = b
                q = 1 - b

                @pl.when(i + 1 < u_per_w)
                def _():
                    wait_idx(q)
                    issue_gather(q)

                wait_gather(p)

                @pl.when(i + 2 < u_per_w)
                def _():
                    issue_idx(i + 2, p)

                @pl.when(i >= 2)
                def _():
                    wait_out(p)

                transpose(p)
                issue_out(i, p)

        for p in range(2):
            wait_out(p)

    return k(idxT, W)


def kernel(input, W):
    B, F = input.shape
    D = W.shape[1]
    idxT = input.T
    o = _emb_lookup(idxT, W, B, F, D)
    return jnp.transpose(o, (2, 0, 1))


# speedup vs baseline: 1.3390x; 1.3390x over previous
import functools

import jax
import jax.numpy as jnp
from jax import lax
from jax.experimental import pallas as pl
from jax.experimental.pallas import tpu as pltpu
from jax.experimental.pallas import tpu_sc as plsc

NC = 2
NS = 16
NW = NC * NS
BZ = 256
L = 16


@functools.partial(jax.jit, static_argnames=("B", "F", "D"))
def _emb_lookup(idxT, W, B, F, D):
    n_blk = B // BZ
    n_units = F * n_blk
    u_per_w = n_units // NW
    assert n_units % NW == 0 and u_per_w % 2 == 0

    mesh = plsc.VectorSubcoreMesh(
        core_axis_name="c", subcore_axis_name="s",
        num_cores=NC, num_subcores=NS,
    )

    @functools.partial(
        pl.kernel,
        out_type=jax.ShapeDtypeStruct((F, D, B), jnp.float32),
        mesh=mesh,
        scratch_types=[
            [pltpu.VMEM((BZ,), jnp.int32)] * 2,
            [pltpu.VMEM((BZ, D), jnp.float32)] * 2,
            [pltpu.VMEM((D, BZ), jnp.float32)] * 2,
            [pltpu.SemaphoreType.DMA] * 2,
            [pltpu.SemaphoreType.DMA] * 2,
            [pltpu.SemaphoreType.DMA] * 2,
        ],
        compiler_params=pltpu.CompilerParams(
            use_tc_tiling_on_sc=False, needs_layout_passes=False
        ),
    )
    def k(idx_hbm, table_hbm, out_hbm, idxs, rows, bufTs, isems, gsems, osems):
        cid = lax.axis_index("c")
        sid = lax.axis_index("s")
        wid = sid * NC + cid
        u0 = wid * u_per_w

        def unit_fb(i):
            u = u0 + i
            f = lax.shift_right_logical(u, 6)
            b0 = pl.multiple_of(
                lax.shift_left(lax.bitwise_and(u, n_blk - 1), 8), BZ
            )
            return f, b0

        def issue_idx(i, p):
            f, b0 = unit_fb(i)
            pltpu.async_copy(idx_hbm.at[f, pl.ds(b0, BZ)], idxs[p], isems[p])

        def wait_idx(p):
            pltpu.make_async_copy(
                idx_hbm.at[0, pl.ds(0, BZ)], idxs[p], isems[p]
            ).wait()

        def issue_gather(p):
            for g in range(BZ // 128):
                pltpu.async_copy(
                    table_hbm.at[idxs[p].at[pl.ds(g * 128, 128)]],
                    rows[p].at[pl.ds(g * 128, 128)],
                    gsems[p],
                )

        def wait_gather(p):
            pltpu.make_async_copy(
                table_hbm.at[pl.ds(0, BZ)], rows[p], gsems[p]
            ).wait()

        def issue_out(i, p):
            f, b0 = unit_fb(i)
            pltpu.async_copy(
                bufTs[p], out_hbm.at[f, :, pl.ds(b0, BZ)], osems[p]
            )

        def wait_out(p):
            pltpu.make_async_copy(
                bufTs[p], out_hbm.at[0, :, pl.ds(0, BZ)], osems[p]
            ).wait()

        ii = lax.iota(jnp.int32, L)
        perms = [lax.bitwise_and(ii + k, L - 1) for k in range(L)]

        def transpose(p):
            @pl.loop(0, BZ, step=L)
            def _(r0):
                ridx = ii + r0
                for d0 in range(0, D, L):
                    for k in range(L):
                        didx = perms[k] + d0
                        x = plsc.load_gather(rows[p], [ridx, didx])
                        plsc.store_scatter(bufTs[p], [didx, ridx], x)

        issue_idx(0, 0)
        wait_idx(0)
        issue_gather(0)
        issue_idx(1, 1)

        @pl.loop(0, u_per_w, step=2)
        def _(o):
            for b in range(2):
                i = o + b
                p = b
                q = 1 - b

                @pl.when(i + 1 < u_per_w)
                def _():
                    wait_idx(q)
                    issue_gather(q)

                wait_gather(p)

                @pl.when(i + 2 < u_per_w)
                def _():
                    issue_idx(i + 2, p)

                @pl.when(i >= 2)
                def _():
                    wait_out(p)

                transpose(p)
                issue_out(i, p)

        for p in range(2):
            wait_out(p)

    return k(idxT, W)


def kernel(input, W):
    B, F = input.shape
    D = W.shape[1]
    idxT = input.T
    o = _emb_lookup(idxT, W, B, F, D)
    return jnp.transpose(o, (2, 0, 1))


# speedup vs baseline: 1.5397x; 1.1498x over previous
import functools

import jax
import jax.numpy as jnp
from jax import lax
from jax.experimental import pallas as pl
from jax.experimental.pallas import tpu as pltpu
from jax.experimental.pallas import tpu_sc as plsc

NC = 2
NS = 16
NW = NC * NS
BZ = 256
L = 16


@functools.partial(jax.jit, static_argnames=("B", "F", "D"))
def _emb_lookup(idxT_flat, W2, B, F, D):
    n_blk = B // BZ
    n_units = F * n_blk
    u_per_w = n_units // NW
    assert n_units % NW == 0 and u_per_w % 2 == 0

    mesh = plsc.VectorSubcoreMesh(
        core_axis_name="c", subcore_axis_name="s",
        num_cores=NC, num_subcores=NS,
    )

    @functools.partial(
        pl.kernel,
        out_type=jax.ShapeDtypeStruct((F, D, B), jnp.float32),
        mesh=mesh,
        scratch_types=[
            [pltpu.VMEM((BZ,), jnp.int32)] * 2,
            [pltpu.VMEM((BZ,), jnp.int32)] * 2,
            [pltpu.VMEM((BZ,), jnp.int32)] * 2,
            [pltpu.VMEM((BZ, 2 * D), jnp.float32)] * 2,
            [pltpu.VMEM((D, BZ), jnp.float32)] * 2,
            [pltpu.SemaphoreType.DMA] * 2,
            [pltpu.SemaphoreType.DMA] * 2,
            [pltpu.SemaphoreType.DMA] * 2,
        ],
        compiler_params=pltpu.CompilerParams(
            use_tc_tiling_on_sc=True, needs_layout_passes=False
        ),
    )
    def k(idx_hbm, table_hbm, out_hbm, idxr, idxh, parv, rows, bufTs,
          isems, gsems, osems):
        cid = lax.axis_index("c")
        sid = lax.axis_index("s")
        wid = sid * NC + cid
        u0 = wid * u_per_w

        def unit_fb(i):
            u = u0 + i
            f = lax.shift_right_logical(u, 6)
            b0 = pl.multiple_of(
                lax.shift_left(lax.bitwise_and(u, n_blk - 1), 8), BZ
            )
            return f, b0

        def issue_idx(i, p):
            f, b0 = unit_fb(i)
            pltpu.async_copy(
                idx_hbm.at[pl.ds(f * B + b0, BZ)], idxr[p], isems[p]
            )

        def wait_idx(p):
            pltpu.make_async_copy(
                idx_hbm.at[pl.ds(0, BZ)], idxr[p], isems[p]
            ).wait()

        def prep_idx(p):
            for t in range(BZ // L):
                v = idxr[p][pl.ds(t * L, L)]
                idxh[p][pl.ds(t * L, L)] = lax.shift_right_logical(v, 1)
                parv[p][pl.ds(t * L, L)] = lax.shift_left(
                    lax.bitwise_and(v, 1), 6
                )

        def issue_gather(p):
            for g in range(BZ // 128):
                pltpu.async_copy(
                    table_hbm.at[idxh[p].at[pl.ds(g * 128, 128)]],
                    rows[p].at[pl.ds(g * 128, 128)],
                    gsems[p],
                )

        def wait_gather(p):
            pltpu.make_async_copy(
                table_hbm.at[pl.ds(0, BZ)], rows[p], gsems[p]
            ).wait()

        def issue_out(i, p):
            f, b0 = unit_fb(i)
            pltpu.async_copy(
                bufTs[p], out_hbm.at[f, :, pl.ds(b0, BZ)], osems[p]
            )

        def wait_out(p):
            pltpu.make_async_copy(
                bufTs[p], out_hbm.at[0, :, pl.ds(0, BZ)], osems[p]
            ).wait()

        ii = lax.iota(jnp.int32, L)
        perms = [lax.bitwise_and(ii + k, L - 1) for k in range(L)]

        def transpose(p):
            @pl.loop(0, BZ, step=L)
            def _(r0):
                ridx = ii + r0
                par16 = parv[p][pl.ds(r0, L)]
                for d0 in range(0, D, L):
                    for k in range(L):
                        dst_d = perms[k] + d0
                        x = plsc.load_gather(rows[p], [ridx, dst_d + par16])
                        plsc.store_scatter(bufTs[p], [dst_d, ridx], x)

        issue_idx(0, 0)
        wait_idx(0)
        prep_idx(0)
        issue_gather(0)
        issue_idx(1, 1)

        @pl.loop(0, u_per_w, step=2)
        def _(o):
            for b in range(2):
                i = o + b
                p = b
                q = 1 - b

                @pl.when(i + 1 < u_per_w)
                def _():
                    wait_idx(q)
                    prep_idx(q)
                    issue_gather(q)

                wait_gather(p)

                @pl.when(i + 2 < u_per_w)
                def _():
                    issue_idx(i + 2, p)

                @pl.when(i >= 2)
                def _():
                    wait_out(p)

                transpose(p)
                issue_out(i, p)

        for p in range(2):
            wait_out(p)

    return k(idxT_flat, W2)


def kernel(input, W):
    B, F = input.shape
    D = W.shape[1]
    idxT_flat = input.T.reshape(-1)
    W2 = W.reshape(W.shape[0] // 2, 2 * D)
    o = _emb_lookup(idxT_flat, W2, B, F, D)
    return jnp.transpose(o, (2, 0, 1))
